# Initial kernel scaffold; baseline (speedup 1.0000x reference)
import jax
import jax.numpy as jnp
from jax.experimental import pallas as pl


def kernel(pcd):
    raise NotImplementedError("write your pallas kernel here")


# speedup vs baseline: 9.6610x; 9.6610x over previous
import functools

import jax
import jax.numpy as jnp
from jax.experimental import pallas as pl

NSAMPLE = 16
K_DENOISE = 32
_ROW_BLK = 256
_RANK_BLK = 512


def _two_sum(a, b):
    s = a + b
    bp = s - a
    e = (a - (s - bp)) + (b - bp)
    return s, e


def _sum3_round_once(t0, t1, t2):
    uh, ul = _two_sum(t1, t2)
    th, tl = _two_sum(t0, uh)
    v = tl + ul
    return th + v


def _meanknn_kernel(q_ref, k_ref, out_ref):
    q = q_ref[0]
    k = k_ref[0]
    R = q.shape[0]
    N = k.shape[1]

    qq = q * q
    sq_q = (qq[:, 0:1] + qq[:, 1:2]) + qq[:, 2:3]
    kk = k * k
    sq_k = (kk[0:1, :] + kk[1:2, :]) + kk[2:3, :]

    qv = q.astype(jnp.bfloat16).astype(jnp.float32)
    kv = k.astype(jnp.bfloat16).astype(jnp.float32)
    t0 = qv[:, 0:1] * kv[0:1, :]
    t1 = qv[:, 1:2] * kv[1:2, :]
    t2 = qv[:, 2:3] * kv[2:3, :]
    inner = _sum3_round_once(t0, t1, t2)
    dist = (sq_q - 2.0 * inner) + sq_k

    iota = jax.lax.broadcasted_iota(jnp.int32, (R, N), 1)
    d = dist
    vals = []
    for _ in range(NSAMPLE):
        m = jnp.min(d, axis=1, keepdims=True)
        vals.append(m)
        eq = d == m
        first = jnp.min(jnp.where(eq, iota, N), axis=1, keepdims=True)
        d = jnp.where(iota == first, jnp.inf, d)
    w = vals
    while len(w) > 1:
        h = len(w) // 2
        w = [w[i] + w[i + h] for i in range(h)]
    out_ref[0, 0] = w[0] / float(NSAMPLE)


def _permute_kernel(mdc_ref, mdr_ref, pcd8_ref, out_ref):
    md_col = mdc_ref[0]
    md_row = mdr_ref[0]
    pcd8 = pcd8_ref[0]
    N = md_row.shape[1]
    C = _RANK_BLK

    j_iota = jax.lax.broadcasted_iota(jnp.int32, (1, N), 1)
    p_iota = j_iota
    r_eff = jnp.where(p_iota < K_DENOISE, (N - 1) - p_iota, p_iota)
    r_eff = r_eff.astype(jnp.float32)

    acc = jnp.zeros((8, N), jnp.float32)
    for ci in range(N // C):
        d_i = md_col[ci * C:(ci + 1) * C, 0:1]
        i_gl = jax.lax.broadcasted_iota(jnp.int32, (C, 1), 0) + ci * C
        gt = md_row > d_i
        eq_lt = (md_row == d_i) & (j_iota < i_gl)
        cnt = (gt | eq_lt).astype(jnp.float32)
        rank = jnp.sum(cnt, axis=1, keepdims=True)
        onehot = (rank == r_eff).astype(jnp.float32)
        pc = pcd8[:, ci * C:(ci + 1) * C]
        acc = acc + jax.lax.dot_general(
            pc, onehot, (((1,), (0,)), ((), ())),
            precision=jax.lax.Precision.HIGHEST,
            preferred_element_type=jnp.float32)
    out_ref[0] = acc


@jax.jit
def _denoise(pcd):
    B, N, _ = pcd.shape
    pcd_t = jnp.transpose(pcd, (0, 2, 1))
    nblk = N // _ROW_BLK
    md = pl.pallas_call(
        _meanknn_kernel,
        grid=(B, nblk),
        in_specs=[
            pl.BlockSpec((1, _ROW_BLK, 3), lambda b, r: (b, r, 0)),
            pl.BlockSpec((1, 3, N), lambda b, r: (b, 0, 0)),
        ],
        out_specs=pl.BlockSpec((1, 1, _ROW_BLK, 1), lambda b, r: (b, r, 0, 0)),
        out_shape=jax.ShapeDtypeStruct((B, nblk, _ROW_BLK, 1), jnp.float32),
    )(pcd, pcd_t)
    md_col = md.reshape(B, N, 1)
    md_row = md.reshape(B, 1, N)
    pcd8 = jnp.concatenate(
        [pcd_t, jnp.zeros((B, 5, N), jnp.float32)], axis=1)
    out8 = pl.pallas_call(
        _permute_kernel,
        grid=(B,),
        in_specs=[
            pl.BlockSpec((1, N, 1), lambda b: (b, 0, 0)),
            pl.BlockSpec((1, 1, N), lambda b: (b, 0, 0)),
            pl.BlockSpec((1, 8, N), lambda b: (b, 0, 0)),
        ],
        out_specs=pl.BlockSpec((1, 8, N), lambda b: (b, 0, 0)),
        out_shape=jax.ShapeDtypeStruct((B, 8, N), jnp.float32),
    )(md_col, md_row, pcd8)
    return jnp.swapaxes(out8, 1, 2)[:, :, :3]


def kernel(pcd):
    return _denoise(pcd)


# speedup vs baseline: 10.9607x; 1.1345x over previous
import functools

import jax
import jax.numpy as jnp
from jax.experimental import pallas as pl

NSAMPLE = 16
K_DENOISE = 32
_ROW_BLK = 256
_RANK_BLK = 512


def _two_sum(a, b):
    s = a + b
    bp = s - a
    e = (a - (s - bp)) + (b - bp)
    return s, e


def _sum3_round_once(t0, t1, t2):
    uh, ul = _two_sum(t1, t2)
    th, tl = _two_sum(t0, uh)
    v = tl + ul
    return th + v


def _meanknn_kernel(q_ref, k_ref, out_ref):
    q = q_ref[0]
    k = k_ref[0]
    R = q.shape[0]
    N = k.shape[1]

    qq = q * q
    sq_q = (qq[:, 0:1] + qq[:, 1:2]) + qq[:, 2:3]
    kk = k * k
    sq_k = (kk[0:1, :] + kk[1:2, :]) + kk[2:3, :]

    qv = q.astype(jnp.bfloat16).astype(jnp.float32)
    kv = k.astype(jnp.bfloat16).astype(jnp.float32)
    t0 = qv[:, 0:1] * kv[0:1, :]
    t1 = qv[:, 1:2] * kv[1:2, :]
    t2 = qv[:, 2:3] * kv[2:3, :]
    inner = _sum3_round_once(t0, t1, t2)
    dist = (sq_q - 2.0 * inner) + sq_k

    H = N // 2
    a = dist[:, :H]
    b = dist[:, H:]
    v = jnp.minimum(a, b)
    e = jnp.maximum(a, b)
    cnt = jnp.zeros((R, H), jnp.int32)
    iota = jax.lax.broadcasted_iota(jnp.int32, (R, H), 1)
    vals = []
    for _ in range(NSAMPLE):
        m = jnp.min(v, axis=1, keepdims=True)
        vals.append(m)
        eq = v == m
        first = jnp.min(jnp.where(eq, iota, H), axis=1, keepdims=True)
        mask = iota == first
        nxt = jnp.where(cnt == 0, e, jnp.inf)
        v = jnp.where(mask, nxt, v)
        cnt = cnt + mask.astype(jnp.int32)
    w = vals
    while len(w) > 1:
        h = len(w) // 2
        w = [w[i] + w[i + h] for i in range(h)]
    out_ref[0, 0] = w[0] / float(NSAMPLE)


def _permute_kernel(mdc_ref, mdr_ref, pcd8_ref, out_ref):
    md_col = mdc_ref[0]
    md_row = mdr_ref[0]
    pcd8 = pcd8_ref[0]
    N = md_row.shape[1]
    C = _RANK_BLK

    j_iota = jax.lax.broadcasted_iota(jnp.int32, (1, N), 1)
    p_iota = j_iota
    r_eff = jnp.where(p_iota < K_DENOISE, (N - 1) - p_iota, p_iota)
    r_eff = r_eff.astype(jnp.float32)

    acc = jnp.zeros((8, N), jnp.float32)
    for ci in range(N // C):
        d_i = md_col[ci * C:(ci + 1) * C, 0:1]
        i_gl = jax.lax.broadcasted_iota(jnp.int32, (C, 1), 0) + ci * C
        gt = md_row > d_i
        eq_lt = (md_row == d_i) & (j_iota < i_gl)
        cnt = (gt | eq_lt).astype(jnp.float32)
        rank = jnp.sum(cnt, axis=1, keepdims=True)
        onehot = (rank == r_eff).astype(jnp.float32)
        pc = pcd8[:, ci * C:(ci + 1) * C]
        acc = acc + jax.lax.dot_general(
            pc, onehot, (((1,), (0,)), ((), ())),
            precision=jax.lax.Precision.HIGHEST,
            preferred_element_type=jnp.float32)
    out_ref[0] = acc


@jax.jit
def _denoise(pcd):
    B, N, _ = pcd.shape
    pcd_t = jnp.transpose(pcd, (0, 2, 1))
    nblk = N // _ROW_BLK
    md = pl.pallas_call(
        _meanknn_kernel,
        grid=(B, nblk),
        in_specs=[
            pl.BlockSpec((1, _ROW_BLK, 3), lambda b, r: (b, r, 0)),
            pl.BlockSpec((1, 3, N), lambda b, r: (b, 0, 0)),
        ],
        out_specs=pl.BlockSpec((1, 1, _ROW_BLK, 1), lambda b, r: (b, r, 0, 0)),
        out_shape=jax.ShapeDtypeStruct((B, nblk, _ROW_BLK, 1), jnp.float32),
    )(pcd, pcd_t)
    md_col = md.reshape(B, N, 1)
    md_row = md.reshape(B, 1, N)
    pcd8 = jnp.concatenate(
        [pcd_t, jnp.zeros((B, 5, N), jnp.float32)], axis=1)
    out8 = pl.pallas_call(
        _permute_kernel,
        grid=(B,),
        in_specs=[
            pl.BlockSpec((1, N, 1), lambda b: (b, 0, 0)),
            pl.BlockSpec((1, 1, N), lambda b: (b, 0, 0)),
            pl.BlockSpec((1, 8, N), lambda b: (b, 0, 0)),
        ],
        out_specs=pl.BlockSpec((1, 8, N), lambda b: (b, 0, 0)),
        out_shape=jax.ShapeDtypeStruct((B, 8, N), jnp.float32),
    )(md_col, md_row, pcd8)
    return jnp.swapaxes(out8, 1, 2)[:, :, :3]


def kernel(pcd):
    return _denoise(pcd)


# speedup vs baseline: 13.5637x; 1.2375x over previous
import functools

import jax
import jax.numpy as jnp
from jax.experimental import pallas as pl

NSAMPLE = 16
K_DENOISE = 32
_ROW_BLK = 256
_RANK_BLK = 512


def _two_sum(a, b):
    s = a + b
    bp = s - a
    e = (a - (s - bp)) + (b - bp)
    return s, e


def _sum3_round_once(t0, t1, t2):
    uh, ul = _two_sum(t1, t2)
    th, tl = _two_sum(t0, uh)
    v = tl + ul
    return th + v


def _meanknn_kernel(q_ref, k_ref, out_ref):
    q = q_ref[0]
    k = k_ref[0]
    R = q.shape[0]
    N = k.shape[1]

    qq = q * q
    sq_q = (qq[:, 0:1] + qq[:, 1:2]) + qq[:, 2:3]
    kk = k * k
    sq_k = (kk[0:1, :] + kk[1:2, :]) + kk[2:3, :]

    qv = q.astype(jnp.bfloat16).astype(jnp.float32)
    kv = k.astype(jnp.bfloat16).astype(jnp.float32)
    t0 = qv[:, 0:1] * kv[0:1, :]
    t1 = qv[:, 1:2] * kv[1:2, :]
    t2 = qv[:, 2:3] * kv[2:3, :]
    inner = _sum3_round_once(t0, t1, t2)
    dist = (sq_q - 2.0 * inner) + sq_k

    Q = N // 4
    a = dist[:, :Q]
    b = dist[:, Q:2 * Q]
    c = dist[:, 2 * Q:3 * Q]
    e = dist[:, 3 * Q:]
    m1 = jnp.minimum(a, b)
    x1 = jnp.maximum(a, b)
    m2 = jnp.minimum(c, e)
    x2 = jnp.maximum(c, e)
    l0 = jnp.minimum(m1, m2)
    t = jnp.maximum(m1, m2)
    l3 = jnp.maximum(x1, x2)
    u = jnp.minimum(x1, x2)
    l1 = jnp.minimum(t, u)
    l2 = jnp.maximum(t, u)
    v = l0
    cnt = jnp.zeros((R, Q), jnp.int32)
    iota = jax.lax.broadcasted_iota(jnp.int32, (R, Q), 1)
    vals = []
    for _ in range(NSAMPLE):
        m = jnp.min(v, axis=1, keepdims=True)
        vals.append(m)
        eq = v == m
        first = jnp.min(jnp.where(eq, iota, Q), axis=1, keepdims=True)
        mask = iota == first
        nxt = jnp.where(cnt == 0, l1,
                        jnp.where(cnt == 1, l2,
                                  jnp.where(cnt == 2, l3, jnp.inf)))
        v = jnp.where(mask, nxt, v)
        cnt = cnt + mask.astype(jnp.int32)
    w = vals
    while len(w) > 1:
        h = len(w) // 2
        w = [w[i] + w[i + h] for i in range(h)]
    out_ref[0, 0] = w[0] / float(NSAMPLE)


def _permute_kernel(mdc_ref, mdr_ref, pcd8_ref, out_ref):
    md_col = mdc_ref[0]
    md_row = mdr_ref[0]
    pcd8 = pcd8_ref[0]
    N = md_row.shape[1]
    C = _RANK_BLK

    j_iota = jax.lax.broadcasted_iota(jnp.int32, (1, N), 1)
    p_iota = j_iota
    r_eff = jnp.where(p_iota < K_DENOISE, (N - 1) - p_iota, p_iota)
    r_eff = r_eff.astype(jnp.float32)

    acc = jnp.zeros((8, N), jnp.float32)
    for ci in range(N // C):
        d_i = md_col[ci * C:(ci + 1) * C, 0:1]
        i_gl = jax.lax.broadcasted_iota(jnp.int32, (C, 1), 0) + ci * C
        gt = md_row > d_i
        eq_lt = (md_row == d_i) & (j_iota < i_gl)
        cnt = (gt | eq_lt).astype(jnp.float32)
        rank = jnp.sum(cnt, axis=1, keepdims=True)
        onehot = (rank == r_eff).astype(jnp.float32)
        pc = pcd8[:, ci * C:(ci + 1) * C]
        acc = acc + jax.lax.dot_general(
            pc, onehot, (((1,), (0,)), ((), ())),
            precision=jax.lax.Precision.HIGHEST,
            preferred_element_type=jnp.float32)
    out_ref[0] = acc


@jax.jit
def _denoise(pcd):
    B, N, _ = pcd.shape
    pcd_t = jnp.transpose(pcd, (0, 2, 1))
    nblk = N // _ROW_BLK
    md = pl.pallas_call(
        _meanknn_kernel,
        grid=(B, nblk),
        in_specs=[
            pl.BlockSpec((1, _ROW_BLK, 3), lambda b, r: (b, r, 0)),
            pl.BlockSpec((1, 3, N), lambda b, r: (b, 0, 0)),
        ],
        out_specs=pl.BlockSpec((1, 1, _ROW_BLK, 1), lambda b, r: (b, r, 0, 0)),
        out_shape=jax.ShapeDtypeStruct((B, nblk, _ROW_BLK, 1), jnp.float32),
    )(pcd, pcd_t)
    md_col = md.reshape(B, N, 1)
    md_row = md.reshape(B, 1, N)
    pcd8 = jnp.concatenate(
        [pcd_t, jnp.zeros((B, 5, N), jnp.float32)], axis=1)
    out8 = pl.pallas_call(
        _permute_kernel,
        grid=(B,),
        in_specs=[
            pl.BlockSpec((1, N, 1), lambda b: (b, 0, 0)),
            pl.BlockSpec((1, 1, N), lambda b: (b, 0, 0)),
            pl.BlockSpec((1, 8, N), lambda b: (b, 0, 0)),
        ],
        out_specs=pl.BlockSpec((1, 8, N), lambda b: (b, 0, 0)),
        out_shape=jax.ShapeDtypeStruct((B, 8, N), jnp.float32),
    )(md_col, md_row, pcd8)
    return jnp.swapaxes(out8, 1, 2)[:, :, :3]


def kernel(pcd):
    return _denoise(pcd)


# speedup vs baseline: 14.6364x; 1.0791x over previous
import functools

import jax
import jax.numpy as jnp
from jax.experimental import pallas as pl

NSAMPLE = 16
K_DENOISE = 32
_ROW_BLK = 256
_RANK_BLK = 512


def _two_sum(a, b):
    s = a + b
    bp = s - a
    e = (a - (s - bp)) + (b - bp)
    return s, e


def _sum3_round_once(t0, t1, t2):
    uh, ul = _two_sum(t1, t2)
    th, tl = _two_sum(t0, uh)
    v = tl + ul
    return th + v


def _meanknn_kernel(q_ref, k_ref, out_ref):
    q = q_ref[0]
    k = k_ref[0]
    R = q.shape[0]
    N = k.shape[1]

    qq = q * q
    sq_q = (qq[:, 0:1] + qq[:, 1:2]) + qq[:, 2:3]
    kk = k * k
    sq_k = (kk[0:1, :] + kk[1:2, :]) + kk[2:3, :]

    qv = q.astype(jnp.bfloat16).astype(jnp.float32)
    kv = k.astype(jnp.bfloat16).astype(jnp.float32)
    t0 = qv[:, 0:1] * kv[0:1, :]
    t1 = qv[:, 1:2] * kv[1:2, :]
    t2 = qv[:, 2:3] * kv[2:3, :]
    inner = _sum3_round_once(t0, t1, t2)
    dist = (sq_q - 2.0 * inner) + sq_k

    def sort4(a, b, c, e):
        m1 = jnp.minimum(a, b)
        x1 = jnp.maximum(a, b)
        m2 = jnp.minimum(c, e)
        x2 = jnp.maximum(c, e)
        l0 = jnp.minimum(m1, m2)
        t = jnp.maximum(m1, m2)
        l3 = jnp.maximum(x1, x2)
        u = jnp.minimum(x1, x2)
        return l0, jnp.minimum(t, u), jnp.maximum(t, u), l3

    Q = N // 8
    ch = [dist[:, i * Q:(i + 1) * Q] for i in range(8)]
    a4 = sort4(ch[0], ch[1], ch[2], ch[3])
    b4 = sort4(ch[4], ch[5], ch[6], ch[7])
    s = [a4[0], a4[1], a4[2], a4[3], b4[3], b4[2], b4[1], b4[0]]
    for dstep in (4, 2, 1):
        ns = list(s)
        for i in range(8):
            if (i & dstep) == 0 and (i + dstep) < 8 and ((i ^ (i + dstep)) == dstep):
                lo = jnp.minimum(s[i], s[i + dstep])
                hi = jnp.maximum(s[i], s[i + dstep])
                ns[i], ns[i + dstep] = lo, hi
        s = ns
    v = s[0]
    cnt = jnp.zeros((R, Q), jnp.int32)
    iota = jax.lax.broadcasted_iota(jnp.int32, (R, Q), 1)
    vals = []
    for _ in range(NSAMPLE):
        m = jnp.min(v, axis=1, keepdims=True)
        vals.append(m)
        eq = v == m
        first = jnp.min(jnp.where(eq, iota, Q), axis=1, keepdims=True)
        mask = iota == first
        nxt = jnp.where(cnt < 3,
                        jnp.where(cnt == 0, s[1],
                                  jnp.where(cnt == 1, s[2], s[3])),
                        jnp.where(cnt < 5,
                                  jnp.where(cnt == 3, s[4], s[5]),
                                  jnp.where(cnt == 5, s[6],
                                            jnp.where(cnt == 6, s[7],
                                                      jnp.inf))))
        v = jnp.where(mask, nxt, v)
        cnt = cnt + mask.astype(jnp.int32)
    w = vals
    while len(w) > 1:
        h = len(w) // 2
        w = [w[i] + w[i + h] for i in range(h)]
    out_ref[0, 0] = w[0] / float(NSAMPLE)


def _permute_kernel(mdc_ref, mdr_ref, pcd8_ref, out_ref):
    md_col = mdc_ref[0]
    md_row = mdr_ref[0]
    pcd8 = pcd8_ref[0]
    N = md_row.shape[1]
    C = _RANK_BLK

    j_iota = jax.lax.broadcasted_iota(jnp.int32, (1, N), 1)
    p_iota = j_iota
    r_eff = jnp.where(p_iota < K_DENOISE, (N - 1) - p_iota, p_iota)
    r_eff = r_eff.astype(jnp.float32)

    acc = jnp.zeros((8, N), jnp.float32)
    for ci in range(N // C):
        d_i = md_col[ci * C:(ci + 1) * C, 0:1]
        i_gl = jax.lax.broadcasted_iota(jnp.int32, (C, 1), 0) + ci * C
        gt = md_row > d_i
        eq_lt = (md_row == d_i) & (j_iota < i_gl)
        cnt = (gt | eq_lt).astype(jnp.float32)
        rank = jnp.sum(cnt, axis=1, keepdims=True)
        onehot = (rank == r_eff).astype(jnp.float32)
        pc = pcd8[:, ci * C:(ci + 1) * C]
        acc = acc + jax.lax.dot_general(
            pc, onehot, (((1,), (0,)), ((), ())),
            precision=jax.lax.Precision.HIGHEST,
            preferred_element_type=jnp.float32)
    out_ref[0] = acc


@jax.jit
def _denoise(pcd):
    B, N, _ = pcd.shape
    pcd_t = jnp.transpose(pcd, (0, 2, 1))
    nblk = N // _ROW_BLK
    md = pl.pallas_call(
        _meanknn_kernel,
        grid=(B, nblk),
        in_specs=[
            pl.BlockSpec((1, _ROW_BLK, 3), lambda b, r: (b, r, 0)),
            pl.BlockSpec((1, 3, N), lambda b, r: (b, 0, 0)),
        ],
        out_specs=pl.BlockSpec((1, 1, _ROW_BLK, 1), lambda b, r: (b, r, 0, 0)),
        out_shape=jax.ShapeDtypeStruct((B, nblk, _ROW_BLK, 1), jnp.float32),
    )(pcd, pcd_t)
    md_col = md.reshape(B, N, 1)
    md_row = md.reshape(B, 1, N)
    pcd8 = jnp.concatenate(
        [pcd_t, jnp.zeros((B, 5, N), jnp.float32)], axis=1)
    out8 = pl.pallas_call(
        _permute_kernel,
        grid=(B,),
        in_specs=[
            pl.BlockSpec((1, N, 1), lambda b: (b, 0, 0)),
            pl.BlockSpec((1, 1, N), lambda b: (b, 0, 0)),
            pl.BlockSpec((1, 8, N), lambda b: (b, 0, 0)),
        ],
        out_specs=pl.BlockSpec((1, 8, N), lambda b: (b, 0, 0)),
        out_shape=jax.ShapeDtypeStruct((B, 8, N), jnp.float32),
    )(md_col, md_row, pcd8)
    return jnp.swapaxes(out8, 1, 2)[:, :, :3]


def kernel(pcd):
    return _denoise(pcd)
